# baseline (speedup 1.0000x reference)
import functools

import jax
import jax.numpy as jnp
from jax import lax
from jax.experimental import pallas as pl
from jax.experimental.pallas import tpu as pltpu
from jax.experimental.pallas import tpu_sc as plsc

_B, _T, _D = 64, 576, 64
_N = _B * _T
_K = 1024

_H = 2
_NH = _N // _H

_R = 1024
_STEPS = _NH // _R

_NC, _NS = 2, 16
_NW = _NC * _NS
_ROWS_PER_TILE = _NH // _NW
_CHUNKS = [128, 128, 128, 128, 64]
assert sum(_CHUNKS) == _ROWS_PER_TILE


def _tc_body(z_ref, cb_ref, idx_ref, loss_ref, cbn_ref, ids_ref):
    i = pl.program_id(0)
    zb = z_ref[...]
    rn = jnp.sum(zb * zb, axis=1, keepdims=True)

    @pl.when(i == 0)
    def _init():
        cb = cb_ref[...]
        cbn_ref[...] = jnp.sum(cb * cb, axis=1)[None, :]
        ids_ref[...] = lax.broadcasted_iota(
            jnp.int32, (_R, _K), 1).astype(jnp.float32)
        loss_ref[0, 0] = 0.0

    dots = lax.dot_general(zb, cb_ref[...], (((1,), (1,)), ((), ())),
                           preferred_element_type=jnp.float32)
    d2 = rn - 2.0 * dots + cbn_ref[...]
    m = jnp.min(d2, axis=1)
    idxf = jnp.min(jnp.where(d2 == m[:, None], ids_ref[...], float(_K)), axis=1)
    idx_ref[...] = idxf.astype(jnp.int32)
    loss_ref[0, 0] += jnp.sum(m)


def _make_tc_call(half):
    base = half * _STEPS
    return pl.pallas_call(
        _tc_body,
        grid=(_STEPS,),
        in_specs=[
            pl.BlockSpec((_R, _D), lambda i: (i + base, 0)),
            pl.BlockSpec((_K, _D), lambda i: (0, 0)),
        ],
        out_specs=[
            pl.BlockSpec((_R,), lambda i: (i,)),
            pl.BlockSpec(memory_space=pltpu.SMEM, block_shape=(1, 1),
                         index_map=lambda i: (0, 0)),
        ],
        out_shape=[
            jax.ShapeDtypeStruct((_NH,), jnp.int32),
            jax.ShapeDtypeStruct((1, 1), jnp.float32),
        ],
        scratch_shapes=[pltpu.VMEM((1, _K), jnp.float32),
                        pltpu.VMEM((_R, _K), jnp.float32)],
    )


_tc_calls = [_make_tc_call(h) for h in range(_H)]


@functools.cache
def _make_sc_gather():
    mesh = plsc.VectorSubcoreMesh(core_axis_name="c", subcore_axis_name="s")

    @functools.partial(
        pl.kernel,
        mesh=mesh,
        out_type=jax.ShapeDtypeStruct((_NH, _D), jnp.float32),
        scratch_types=[
            pltpu.VMEM((_ROWS_PER_TILE,), jnp.int32),
            pltpu.VMEM((_ROWS_PER_TILE, _D), jnp.float32),
            pltpu.SemaphoreType.DMA,
        ],
        compiler_params=pltpu.CompilerParams(use_tc_tiling_on_sc=False),
    )
    def _sc_gather(cb_hbm, idx_hbm, out_hbm, idx_v, rows_v, sem):
        wid = lax.axis_index("s") * _NC + lax.axis_index("c")
        base = wid * _ROWS_PER_TILE
        pltpu.sync_copy(idx_hbm.at[pl.ds(base, _ROWS_PER_TILE)], idx_v)
        copies = []
        off = 0
        for c in _CHUNKS:
            copies.append(pltpu.async_copy(
                cb_hbm.at[idx_v.at[pl.ds(off, c)]],
                rows_v.at[pl.ds(off, c), :],
                sem,
            ))
            off += c
        for cp in copies:
            cp.wait()
        pltpu.sync_copy(rows_v, out_hbm.at[pl.ds(base, _ROWS_PER_TILE)])

    return _sc_gather


def kernel(z, codebook):
    B, T, D = z.shape
    flat = z.reshape(_N, D)
    gather = _make_sc_gather()
    idx_halves, loss_halves, q_halves = [], [], []
    for h in range(_H):
        idx_h, loss_h = _tc_calls[h](flat, codebook)
        q_halves.append(gather(codebook, idx_h))
        idx_halves.append(idx_h)
        loss_halves.append(loss_h)
    st = jnp.concatenate(q_halves, axis=0).reshape(B, T, D)
    loss_sum = loss_halves[0] + loss_halves[1]
    vq_loss = loss_sum.reshape(()) * (1.25 / (_N * _D))
    idx = jnp.concatenate(idx_halves, axis=0).reshape(B, T)
    return st, vq_loss, idx
